# baseline (speedup 1.0000x reference)
import functools
import math

import jax
import jax.numpy as jnp
from jax import lax
from jax.experimental import pallas as pl
from jax.experimental.pallas import tpu as pltpu
from jax.experimental.pallas import tpu_sc as plsc

_N_SUBCORES = 16
_LANES = 16


def _logits_kernel(x_ref, wg_ref, out_ref):
    out_ref[...] = lax.dot(x_ref[...], wg_ref[...],
                           preferred_element_type=jnp.float32)


def _combine_kernel(x_ref, coef_ref, wcat_ref, sel_ref, fold_ref, be_ref,
                    out_ref):
    x = x_ref[...]
    y_all = lax.dot(x, wcat_ref[...],
                    preferred_element_type=jnp.float32)
    coef_b = lax.dot(coef_ref[...], sel_ref[...],
                     preferred_element_type=jnp.float32)
    z = coef_b * (y_all + be_ref[...])
    out_ref[...] = lax.dot(z, fold_ref[...],
                           preferred_element_type=jnp.float32)


def _sc_routing_body(logits_hbm, coef_hbm, lg_v, idx_v, gate_v, coef_v,
                     row_v, allcnt_v, shared, *, tok_per_tile: int,
                     n_experts: int, capacity: int):
    core = lax.axis_index("c")
    sid = lax.axis_index("s")
    E = n_experts
    n_groups = tok_per_tile // _LANES
    span = tok_per_tile * E
    lanes = lax.iota(jnp.int32, _LANES)

    @pl.when(core == 0)
    def _():
        base_hbm = sid * span
        pltpu.sync_copy(logits_hbm.at[pl.ds(base_hbm, span)], lg_v)

        def body_a(g, cnts):
            base = g * (_LANES * E)
            ls = [plsc.load_gather(lg_v, [base + lanes * E + e])
                  for e in range(E)]
            m = ls[0]
            for e in range(1, E):
                m = jnp.maximum(m, ls[e])
            idx = jnp.full((_LANES,), E - 1, jnp.int32)
            for e in range(E - 2, -1, -1):
                idx = jnp.where(ls[e] == m, e, idx)
            s = jnp.exp(ls[0] - m)
            for e in range(1, E):
                s = s + jnp.exp(ls[e] - m)
            gate = 1.0 / s
            idx_v[pl.ds(g * _LANES, _LANES)] = idx
            gate_v[pl.ds(g * _LANES, _LANES)] = gate
            new = []
            for e in range(E):
                c = plsc.all_reduce_population_count(idx == e)
                new.append(cnts[e] + c)
            return tuple(new)

        zero = jnp.zeros((_LANES,), jnp.int32)
        cnts = lax.fori_loop(0, n_groups, body_a, (zero,) * E)

        cvec = jnp.zeros((_LANES,), jnp.int32)
        for e in range(E):
            cvec = jnp.where(lanes == e, cnts[e], cvec)
        row_v[...] = cvec
        pltpu.sync_copy(row_v, shared.at[sid])
        plsc.subcore_barrier()
        pltpu.sync_copy(shared, allcnt_v)

        sid_vec = jnp.zeros((_LANES,), jnp.int32) + sid
        offs = []
        for e in range(E):
            col = plsc.load_gather(
                allcnt_v, [lanes, jnp.zeros((_LANES,), jnp.int32) + e])
            exc = plsc.cumsum(col) - col
            row_v[...] = exc
            offs.append(plsc.load_gather(row_v, [sid_vec]))

        def body_b(g, runs):
            tbase = g * _LANES
            idx = idx_v[pl.ds(tbase, _LANES)]
            gate = gate_v[pl.ds(tbase, _LANES)]
            new = []
            for e in range(E):
                me = idx == e
                pre = plsc.cumsum(jnp.where(me, 1, 0))
                rank = runs[e] + pre - 1
                keep = me & (rank < capacity)
                vals = jnp.where(keep, gate, 0.0)
                plsc.store_scatter(coef_v, [tbase * E + lanes * E + e], vals)
                tot = plsc.all_reduce_population_count(me)
                new.append(runs[e] + tot)
            return tuple(new)

        lax.fori_loop(0, n_groups, body_b, tuple(offs))
        pltpu.sync_copy(coef_v, coef_hbm.at[pl.ds(base_hbm, span)])


def _sc_routing(logits_flat, *, T: int, n_experts: int, capacity: int):
    E = n_experts
    tok_per_tile = T // _N_SUBCORES
    span = tok_per_tile * E
    mesh = plsc.VectorSubcoreMesh(core_axis_name="c", subcore_axis_name="s")
    body = functools.partial(_sc_routing_body, tok_per_tile=tok_per_tile,
                             n_experts=E, capacity=capacity)
    return pl.kernel(
        body,
        out_type=jax.ShapeDtypeStruct((T * E,), jnp.float32),
        mesh=mesh,
        compiler_params=pltpu.CompilerParams(
            use_tc_tiling_on_sc=False, needs_layout_passes=False),
        scratch_types=[
            pltpu.VMEM((span,), jnp.float32),
            pltpu.VMEM((tok_per_tile,), jnp.int32),
            pltpu.VMEM((tok_per_tile,), jnp.float32),
            pltpu.VMEM((span,), jnp.float32),
            pltpu.VMEM((_LANES,), jnp.int32),
            pltpu.VMEM((_N_SUBCORES, _LANES), jnp.int32),
            pltpu.VMEM_SHARED((_N_SUBCORES, _LANES), jnp.int32),
        ],
    )(logits_flat)


def kernel(inputs, Wg, We, be):
    d = inputs.shape[-1]
    E = Wg.shape[1]
    x = inputs.reshape(-1, d)
    T = x.shape[0]
    capacity = int(math.ceil(T / E))

    B = 4096
    assert T % B == 0
    n_blocks = T // B

    logits = pl.pallas_call(
        _logits_kernel,
        grid=(n_blocks,),
        in_specs=[
            pl.BlockSpec((B, d), lambda i: (i, 0)),
            pl.BlockSpec((d, E), lambda i: (0, 0)),
        ],
        out_specs=pl.BlockSpec((B, E), lambda i: (i, 0)),
        out_shape=jax.ShapeDtypeStruct((T, E), jnp.float32),
    )(x, Wg)

    coef = _sc_routing(logits.reshape(-1), T=T, n_experts=E,
                       capacity=capacity).reshape(T, E)

    wcat = We.transpose(1, 0, 2).reshape(d, E * d)
    sel = jnp.repeat(jnp.eye(E, dtype=jnp.float32), d, axis=1)
    fold = jnp.tile(jnp.eye(d, dtype=jnp.float32), (E, 1))
    be_flat = be.reshape(1, E * d)

    out = pl.pallas_call(
        _combine_kernel,
        grid=(n_blocks,),
        in_specs=[
            pl.BlockSpec((B, d), lambda i: (i, 0)),
            pl.BlockSpec((B, E), lambda i: (i, 0)),
            pl.BlockSpec((d, E * d), lambda i: (0, 0)),
            pl.BlockSpec((E, E * d), lambda i: (0, 0)),
            pl.BlockSpec((E * d, d), lambda i: (0, 0)),
            pl.BlockSpec((1, E * d), lambda i: (0, 0)),
        ],
        out_specs=pl.BlockSpec((B, d), lambda i: (i, 0)),
        out_shape=jax.ShapeDtypeStruct((T, d), jnp.float32),
    )(x, coef, wcat, sel, fold, be_flat)
    return out.reshape(inputs.shape)


# speedup vs baseline: 1.0012x; 1.0012x over previous
import functools
import math

import jax
import jax.numpy as jnp
from jax import lax
from jax.experimental import pallas as pl
from jax.experimental.pallas import tpu as pltpu
from jax.experimental.pallas import tpu_sc as plsc

_N_SUBCORES = 16
_LANES = 16


def _logits_kernel(x_ref, wg_ref, out_ref):
    out_ref[...] = lax.dot(x_ref[...], wg_ref[...],
                           preferred_element_type=jnp.float32)


def _combine_kernel(x_ref, coef_ref, wcat_ref, sel_ref, fold_ref, be_ref,
                    out_ref):
    x = x_ref[...]
    y_all = lax.dot(x, wcat_ref[...],
                    preferred_element_type=jnp.float32)
    coef_b = lax.dot(coef_ref[...], sel_ref[...],
                     preferred_element_type=jnp.float32)
    z = coef_b * (y_all + be_ref[...])
    out_ref[...] = lax.dot(z, fold_ref[...],
                           preferred_element_type=jnp.float32)


def _sc_routing_body(logits_hbm, coef_hbm, lg_v, idx_v, gate_v, coef_v,
                     row_v, allcnt_v, shared, *, tok_per_tile: int,
                     n_experts: int, capacity: int):
    core = lax.axis_index("c")
    sid = lax.axis_index("s")
    E = n_experts
    n_groups = tok_per_tile // _LANES
    span = tok_per_tile * E
    lanes = lax.iota(jnp.int32, _LANES)

    @pl.when(core == 0)
    def _():
        base_hbm = sid * span
        pltpu.sync_copy(logits_hbm.at[pl.ds(base_hbm, span)], lg_v)

        def body_a(g, cnts):
            base = g * (_LANES * E)
            ls = [plsc.load_gather(lg_v, [base + lanes * E + e])
                  for e in range(E)]
            m = ls[0]
            for e in range(1, E):
                m = jnp.maximum(m, ls[e])
            idx = jnp.full((_LANES,), E - 1, jnp.int32)
            for e in range(E - 2, -1, -1):
                idx = jnp.where(ls[e] == m, e, idx)
            s = jnp.exp(ls[0] - m)
            for e in range(1, E):
                s = s + jnp.exp(ls[e] - m)
            gate = 1.0 / s
            idx_v[pl.ds(g * _LANES, _LANES)] = idx
            gate_v[pl.ds(g * _LANES, _LANES)] = gate
            new = []
            for e in range(E):
                c = plsc.all_reduce_population_count(idx == e)
                new.append(cnts[e] + c)
            return tuple(new)

        zero = jnp.zeros((_LANES,), jnp.int32)
        cnts = lax.fori_loop(0, n_groups, body_a, (zero,) * E)

        cvec = jnp.zeros((_LANES,), jnp.int32)
        for e in range(E):
            cvec = jnp.where(lanes == e, cnts[e], cvec)
        row_v[...] = cvec
        pltpu.sync_copy(row_v, shared.at[sid])
        plsc.subcore_barrier()
        pltpu.sync_copy(shared, allcnt_v)

        sid_vec = jnp.zeros((_LANES,), jnp.int32) + sid
        offs = []
        for e in range(E):
            col = plsc.load_gather(
                allcnt_v, [lanes, jnp.zeros((_LANES,), jnp.int32) + e])
            exc = plsc.cumsum(col) - col
            row_v[...] = exc
            offs.append(plsc.load_gather(row_v, [sid_vec]))

        def body_b(g, runs):
            tbase = g * _LANES
            idx = idx_v[pl.ds(tbase, _LANES)]
            gate = gate_v[pl.ds(tbase, _LANES)]
            new = []
            for e in range(E):
                me = idx == e
                pre = plsc.cumsum(jnp.where(me, 1, 0))
                rank = runs[e] + pre - 1
                keep = me & (rank < capacity)
                vals = jnp.where(keep, gate, 0.0)
                plsc.store_scatter(coef_v, [tbase * E + lanes * E + e], vals)
                tot = plsc.all_reduce_population_count(me)
                new.append(runs[e] + tot)
            return tuple(new)

        lax.fori_loop(0, n_groups, body_b, tuple(offs))
        pltpu.sync_copy(coef_v, coef_hbm.at[pl.ds(base_hbm, span)])


def _sc_routing(logits_flat, *, T: int, n_experts: int, capacity: int):
    E = n_experts
    tok_per_tile = T // _N_SUBCORES
    span = tok_per_tile * E
    mesh = plsc.VectorSubcoreMesh(core_axis_name="c", subcore_axis_name="s")
    body = functools.partial(_sc_routing_body, tok_per_tile=tok_per_tile,
                             n_experts=E, capacity=capacity)
    return pl.kernel(
        body,
        out_type=jax.ShapeDtypeStruct((T * E,), jnp.float32),
        mesh=mesh,
        compiler_params=pltpu.CompilerParams(
            use_tc_tiling_on_sc=False, needs_layout_passes=False),
        scratch_types=[
            pltpu.VMEM((span,), jnp.float32),
            pltpu.VMEM((tok_per_tile,), jnp.int32),
            pltpu.VMEM((tok_per_tile,), jnp.float32),
            pltpu.VMEM((span,), jnp.float32),
            pltpu.VMEM((_LANES,), jnp.int32),
            pltpu.VMEM((_N_SUBCORES, _LANES), jnp.int32),
            pltpu.VMEM_SHARED((_N_SUBCORES, _LANES), jnp.int32),
        ],
    )(logits_flat)


def kernel(inputs, Wg, We, be):
    d = inputs.shape[-1]
    E = Wg.shape[1]
    x = inputs.reshape(-1, d)
    T = x.shape[0]
    capacity = int(math.ceil(T / E))

    B = 4096
    assert T % B == 0
    n_blocks = T // B

    logits = pl.pallas_call(
        _logits_kernel,
        in_specs=[
            pl.BlockSpec((T, d), lambda: (0, 0)),
            pl.BlockSpec((d, E), lambda: (0, 0)),
        ],
        out_specs=pl.BlockSpec((T, E), lambda: (0, 0)),
        out_shape=jax.ShapeDtypeStruct((T, E), jnp.float32),
    )(x, Wg)

    coef = _sc_routing(logits.reshape(-1), T=T, n_experts=E,
                       capacity=capacity).reshape(T, E)

    wcat = We.transpose(1, 0, 2).reshape(d, E * d)
    sel = jnp.repeat(jnp.eye(E, dtype=jnp.float32), d, axis=1)
    fold = jnp.tile(jnp.eye(d, dtype=jnp.float32), (E, 1))
    be_flat = be.reshape(1, E * d)

    out = pl.pallas_call(
        _combine_kernel,
        grid=(n_blocks,),
        in_specs=[
            pl.BlockSpec((B, d), lambda i: (i, 0)),
            pl.BlockSpec((B, E), lambda i: (i, 0)),
            pl.BlockSpec((d, E * d), lambda i: (0, 0)),
            pl.BlockSpec((E, E * d), lambda i: (0, 0)),
            pl.BlockSpec((E * d, d), lambda i: (0, 0)),
            pl.BlockSpec((1, E * d), lambda i: (0, 0)),
        ],
        out_specs=pl.BlockSpec((B, d), lambda i: (i, 0)),
        out_shape=jax.ShapeDtypeStruct((T, d), jnp.float32),
    )(x, coef, wcat, sel, fold, be_flat)
    return out.reshape(inputs.shape)
